# baseline (speedup 1.0000x reference)
import functools

import jax
import jax.numpy as jnp
from jax import lax
from jax.experimental import pallas as pl
from jax.experimental.pallas import tpu as pltpu
from jax.experimental.pallas import tpu_sc as plsc

BATCH = 16384
NF = 32
NW = 32
B_PER_W = BATCH // NW
IDX_CHUNK = 128
N_CHUNKS = B_PER_W // IDX_CHUNK


def _sc_gather_body(uid_hbm, iid_hbm, gu_hbm, gi_hbm, mu_hbm, mi_hbm,
                    gmf_out, mu_out, mi_out,
                    idx_u, idx_i, gu_v, gi_v, mu_v, mi_v, sem):
    wid = lax.axis_index("s") * 2 + lax.axis_index("c")
    base = wid * B_PER_W

    for j in range(N_CHUNKS):
        pltpu.sync_copy(uid_hbm.at[pl.ds(base + j * IDX_CHUNK, IDX_CHUNK)],
                        idx_u.at[j])
        pltpu.sync_copy(iid_hbm.at[pl.ds(base + j * IDX_CHUNK, IDX_CHUNK)],
                        idx_i.at[j])

    copies = []
    for j in range(N_CHUNKS):
        rows = pl.ds(j * IDX_CHUNK, IDX_CHUNK)
        copies.append(pltpu.async_copy(gu_hbm.at[idx_u.at[j]], gu_v.at[rows], sem))
        copies.append(pltpu.async_copy(gi_hbm.at[idx_i.at[j]], gi_v.at[rows], sem))
        copies.append(pltpu.async_copy(mu_hbm.at[idx_u.at[j]], mu_v.at[rows], sem))
        copies.append(pltpu.async_copy(mi_hbm.at[idx_i.at[j]], mi_v.at[rows], sem))
    for c in copies:
        c.wait()

    def mul_row(r, carry):
        for h in range(NF // 16):
            s = pl.ds(h * 16, 16)
            gu_v[r, s] = gu_v[r, s] * gi_v[r, s]
        return carry
    lax.fori_loop(0, B_PER_W, mul_row, 0)

    out_rows = pl.ds(base, B_PER_W)
    pltpu.sync_copy(gu_v, gmf_out.at[out_rows])
    pltpu.sync_copy(mu_v, mu_out.at[out_rows])
    pltpu.sync_copy(mi_v, mi_out.at[out_rows])


def _sc_gather(user_ids, item_ids, gmf_user_emb, gmf_item_emb,
               mlp_user_emb, mlp_item_emb):
    mesh = plsc.VectorSubcoreMesh(core_axis_name="c", subcore_axis_name="s")
    f32 = jnp.float32
    out_type = (
        jax.ShapeDtypeStruct((BATCH, NF), f32),
        jax.ShapeDtypeStruct((BATCH, NF), f32),
        jax.ShapeDtypeStruct((BATCH, NF), f32),
    )
    scratch = [
        pltpu.VMEM((N_CHUNKS, IDX_CHUNK), jnp.int32),
        pltpu.VMEM((N_CHUNKS, IDX_CHUNK), jnp.int32),
        pltpu.VMEM((B_PER_W, NF), f32),
        pltpu.VMEM((B_PER_W, NF), f32),
        pltpu.VMEM((B_PER_W, NF), f32),
        pltpu.VMEM((B_PER_W, NF), f32),
        pltpu.SemaphoreType.DMA,
    ]
    fn = pl.kernel(_sc_gather_body, out_type=out_type, mesh=mesh,
                   scratch_types=scratch,
                   compiler_params=pltpu.CompilerParams(
                       use_tc_tiling_on_sc=False))
    return fn(user_ids, item_ids, gmf_user_emb, gmf_item_emb,
              mlp_user_emb, mlp_item_emb)


BB = 1024


def _tc_mlp_body(gmf_ref, mu_ref, mi_ref, W1_ref, b1_ref, W2_ref, b2_ref,
                 W3_ref, b3_ref, Wo_ref, bo_ref, out_ref):
    f32 = jnp.float32
    mu = mu_ref[...]
    mi = mi_ref[...]
    w1 = W1_ref[...]
    h = jnp.dot(mu, w1[:NF], preferred_element_type=f32)
    h += jnp.dot(mi, w1[NF:], preferred_element_type=f32)
    h = jnp.maximum(h + b1_ref[...], 0.0)
    h = jnp.maximum(jnp.dot(h, W2_ref[...], preferred_element_type=f32)
                    + b2_ref[...], 0.0)
    h = jnp.maximum(jnp.dot(h, W3_ref[...], preferred_element_type=f32)
                    + b3_ref[...], 0.0)
    wo = Wo_ref[...]
    logits = jnp.dot(gmf_ref[...], wo[:NF], preferred_element_type=f32)
    logits += jnp.dot(h, wo[NF:], preferred_element_type=f32)
    logits += bo_ref[...]
    out_ref[...] = jax.nn.sigmoid(logits)


def _tc_mlp(gmf_vec, mu_rows, mi_rows, W1, b1, W2, b2, W3, b3, Wo, bo):
    grid = (BATCH // BB,)
    row_spec = pl.BlockSpec((BB, NF), lambda i: (i, 0))
    full = lambda a: pl.BlockSpec(a.shape, lambda i: (0,) * a.ndim)
    return pl.pallas_call(
        _tc_mlp_body,
        grid=grid,
        in_specs=[row_spec, row_spec, row_spec,
                  full(W1), full(b1), full(W2), full(b2),
                  full(W3), full(b3), full(Wo), full(bo)],
        out_specs=pl.BlockSpec((BB, 1), lambda i: (i, 0)),
        out_shape=jax.ShapeDtypeStruct((BATCH, 1), jnp.float32),
        compiler_params=pltpu.CompilerParams(
            dimension_semantics=("arbitrary",)),
    )(gmf_vec, mu_rows, mi_rows, W1, b1, W2, b2, W3, b3, Wo, bo)


def kernel(user_ids, item_ids, gmf_user_emb, gmf_item_emb, mlp_user_emb,
           mlp_item_emb, W1, b1, W2, b2, W3, b3, Wo, bo):
    gmf_vec, mu_rows, mi_rows = _sc_gather(
        user_ids, item_ids, gmf_user_emb, gmf_item_emb,
        mlp_user_emb, mlp_item_emb)
    b1r = b1.reshape(1, -1)
    b2r = b2.reshape(1, -1)
    b3r = b3.reshape(1, -1)
    bor = bo.reshape(1, -1)
    return _tc_mlp(gmf_vec, mu_rows, mi_rows, W1, b1r, W2, b2r, W3, b3r,
                   Wo, bor)


# speedup vs baseline: 2.3599x; 2.3599x over previous
import functools

import jax
import jax.numpy as jnp
from jax import lax
from jax.experimental import pallas as pl
from jax.experimental.pallas import tpu as pltpu
from jax.experimental.pallas import tpu_sc as plsc

BATCH = 16384
NF = 32
NW = 32
B_PER_W = BATCH // NW
CHUNK = 128
N_CHUNKS = B_PER_W // CHUNK
L = 16


def _sc_gather_body(uid_hbm, iid_hbm, gu_hbm, gi_hbm, mu_hbm, mi_hbm,
                    gmf_out, mu_out, mi_out,
                    ids_u_s, ids_i_s,
                    e_gu, e_gi, e_u, e_i, sem):
    wid = lax.axis_index("s") * 2 + lax.axis_index("c")
    base = wid * B_PER_W

    pltpu.sync_copy(uid_hbm.at[pl.ds(base, B_PER_W)], ids_u_s)
    pltpu.sync_copy(iid_hbm.at[pl.ds(base, B_PER_W)], ids_i_s)

    def chunk_step(k, carry):
        c0 = k * CHUNK

        def issue(g, carry2):
            u_vec = ids_u_s[pl.ds(c0 + g * L, L)]
            i_vec = ids_i_s[pl.ds(c0 + g * L, L)]
            for l in range(L):
                j = g * L + l
                u = u_vec[l]
                i = i_vec[l]
                gu = lax.shift_right_logical(u, 3)
                su = lax.bitwise_and(u, 7)
                gi = lax.shift_right_logical(i, 3)
                si = lax.bitwise_and(i, 7)
                pltpu.make_async_copy(gu_hbm.at[gu, su], e_gu.at[j],
                                      sem).start()
                pltpu.make_async_copy(gi_hbm.at[gi, si], e_gi.at[j],
                                      sem).start()
                pltpu.make_async_copy(mu_hbm.at[gu, su], e_u.at[j],
                                      sem).start()
                pltpu.make_async_copy(mi_hbm.at[gi, si], e_i.at[j],
                                      sem).start()
            return carry2
        lax.fori_loop(0, CHUNK // L, issue, 0)

        rows = pl.ds(base + c0, CHUNK)
        pltpu.make_async_copy(gmf_out.at[rows], e_gu, sem).wait()
        pltpu.make_async_copy(gmf_out.at[rows], e_gi, sem).wait()
        pltpu.make_async_copy(mu_out.at[rows], e_u, sem).wait()
        pltpu.make_async_copy(mi_out.at[rows], e_i, sem).wait()

        def mul_step(m, carry2):
            row = m // (NF // L)
            col = pl.ds((m % (NF // L)) * L, L)
            e_gu[row, col] = e_gu[row, col] * e_gi[row, col]
            return carry2
        lax.fori_loop(0, CHUNK * NF // L, mul_step, 0)

        pltpu.sync_copy(e_gu, gmf_out.at[rows])
        pltpu.sync_copy(e_u, mu_out.at[rows])
        pltpu.sync_copy(e_i, mi_out.at[rows])
        return carry
    lax.fori_loop(0, N_CHUNKS, chunk_step, 0)


def _sc_gather(user_ids, item_ids, gu3, gi3, mu3, mi3):
    mesh = plsc.VectorSubcoreMesh(core_axis_name="c", subcore_axis_name="s")
    f32 = jnp.float32
    i32 = jnp.int32
    out_type = (
        jax.ShapeDtypeStruct((BATCH, NF), f32),
        jax.ShapeDtypeStruct((BATCH, NF), f32),
        jax.ShapeDtypeStruct((BATCH, NF), f32),
    )
    scratch = [
        pltpu.VMEM((B_PER_W,), i32),
        pltpu.VMEM((B_PER_W,), i32),
        pltpu.VMEM((CHUNK, NF), f32),
        pltpu.VMEM((CHUNK, NF), f32),
        pltpu.VMEM((CHUNK, NF), f32),
        pltpu.VMEM((CHUNK, NF), f32),
        pltpu.SemaphoreType.DMA,
    ]
    fn = pl.kernel(_sc_gather_body, out_type=out_type, mesh=mesh,
                   scratch_types=scratch,
                   compiler_params=pltpu.CompilerParams(
                       use_tc_tiling_on_sc=True))
    return fn(user_ids, item_ids, gu3, gi3, mu3, mi3)


BB = 1024


def _tc_mlp_body(gmf_ref, mu_ref, mi_ref, W1_ref, b1_ref, W2_ref, b2_ref,
                 W3_ref, b3_ref, Wo_ref, bo_ref, out_ref):
    f32 = jnp.float32
    w1 = W1_ref[...]
    h = jnp.dot(mu_ref[...], w1[:NF], preferred_element_type=f32)
    h += jnp.dot(mi_ref[...], w1[NF:], preferred_element_type=f32)
    h = jnp.maximum(h + b1_ref[...], 0.0)
    h = jnp.maximum(jnp.dot(h, W2_ref[...], preferred_element_type=f32)
                    + b2_ref[...], 0.0)
    h = jnp.maximum(jnp.dot(h, W3_ref[...], preferred_element_type=f32)
                    + b3_ref[...], 0.0)
    wo = Wo_ref[...]
    logits = jnp.dot(gmf_ref[...], wo[:NF], preferred_element_type=f32)
    logits += jnp.dot(h, wo[NF:], preferred_element_type=f32)
    logits += bo_ref[...]
    out_ref[...] = jax.nn.sigmoid(logits)


def _tc_mlp(gmf_vec, mu_rows, mi_rows, W1, b1, W2, b2, W3, b3, Wo, bo):
    grid = (BATCH // BB,)
    row_spec = pl.BlockSpec((BB, NF), lambda i: (i, 0))
    full = lambda a: pl.BlockSpec(a.shape, lambda i: (0,) * a.ndim)
    return pl.pallas_call(
        _tc_mlp_body,
        grid=grid,
        in_specs=[row_spec, row_spec, row_spec,
                  full(W1), full(b1), full(W2), full(b2),
                  full(W3), full(b3), full(Wo), full(bo)],
        out_specs=pl.BlockSpec((BB, 1), lambda i: (i, 0)),
        out_shape=jax.ShapeDtypeStruct((BATCH, 1), jnp.float32),
        compiler_params=pltpu.CompilerParams(
            dimension_semantics=("arbitrary",)),
    )(gmf_vec, mu_rows, mi_rows, W1, b1, W2, b2, W3, b3, Wo, bo)


def kernel(user_ids, item_ids, gmf_user_emb, gmf_item_emb, mlp_user_emb,
           mlp_item_emb, W1, b1, W2, b2, W3, b3, Wo, bo):
    gu3 = gmf_user_emb.reshape(-1, 8, NF)
    gi3 = gmf_item_emb.reshape(-1, 8, NF)
    mu3 = mlp_user_emb.reshape(-1, 8, NF)
    mi3 = mlp_item_emb.reshape(-1, 8, NF)
    gmf_vec, mu_rows, mi_rows = _sc_gather(
        user_ids, item_ids, gu3, gi3, mu3, mi3)
    b1r = b1.reshape(1, -1)
    b2r = b2.reshape(1, -1)
    b3r = b3.reshape(1, -1)
    return _tc_mlp(gmf_vec, mu_rows, mi_rows, W1, b1r, W2, b2r, W3, b3r,
                   Wo, bor := bo.reshape(1, -1))
